# Initial kernel scaffold; baseline (speedup 1.0000x reference)
import jax
import jax.numpy as jnp
from jax.experimental import pallas as pl


def kernel(edge_index, edge_attrs, emb_weight, W1, b1, W2, b2):
    raise NotImplementedError("write your pallas kernel here")


# speedup vs baseline: 4.9664x; 4.9664x over previous
import functools

import jax
import jax.numpy as jnp
from jax import lax
from jax.experimental import pallas as pl
from jax.experimental.pallas import tpu as pltpu
from jax.experimental.pallas import tpu_sc as plsc

NN = 10000
EE = 160000
DD = 256
HH = 128
LL = 3
NC = 2
NS = 16
RPW = 632
NPAD = NC * NS * RPW // NC
K_ROW = 80
ROW_CHUNKS = EE // NS // K_ROW
K_NAR = 40
NAR_CHUNKS = EE // (NC * NS) // K_NAR
BR = 400


def _sc_mesh():
    return plsc.VectorSubcoreMesh(core_axis_name="c", subcore_axis_name="s")


@functools.partial(
    pl.kernel,
    out_type=jax.ShapeDtypeStruct((NC, NPAD, HH), jnp.float32),
    mesh=_sc_mesh(),
    scratch_types=[
        pltpu.VMEM((K_NAR,), jnp.int32),
        pltpu.VMEM((K_NAR, HH), jnp.float32),
        pltpu.VMEM_SHARED((NPAD, HH), jnp.float32),
    ],
)
def _deg_pass(to_hbm, ones_hbm, znar_hbm, deg_out, tidx_v, ones_v, acc_sh):
    c = lax.axis_index("c")
    s = lax.axis_index("s")
    pltpu.sync_copy(ones_hbm, ones_v)
    pltpu.sync_copy(znar_hbm, acc_sh.at[pl.ds(s * RPW, RPW)])
    plsc.subcore_barrier()
    base = c * (EE // NC) + s * (EE // NC // NS)

    def body(i, carry):
        off = pl.multiple_of(base + i * K_NAR, 8)
        pltpu.sync_copy(to_hbm.at[pl.ds(off, K_NAR)], tidx_v)
        pltpu.sync_copy(ones_v, acc_sh.at[tidx_v], add=True)
        return carry

    lax.fori_loop(0, NAR_CHUNKS, body, 0)
    plsc.subcore_barrier()
    pltpu.sync_copy(acc_sh.at[pl.ds(s * RPW, RPW)],
                    deg_out.at[c, pl.ds(s * RPW, RPW)])


@functools.partial(
    pl.kernel,
    out_type=jax.ShapeDtypeStruct((NC, NPAD, HH), jnp.float32),
    mesh=_sc_mesh(),
    scratch_types=[
        pltpu.VMEM((K_NAR,), jnp.int32),
        pltpu.VMEM((K_NAR,), jnp.int32),
        pltpu.VMEM((K_NAR, HH), jnp.float32),
        pltpu.SemaphoreType.DMA,
        pltpu.VMEM_SHARED((NPAD, HH), jnp.float32),
    ],
)
def _t_pass(fr_hbm, to_hbm, dis_hbm, znar_hbm, t_out,
            fidx_v, tidx_v, rows_v, sem, acc_sh):
    c = lax.axis_index("c")
    s = lax.axis_index("s")
    pltpu.sync_copy(znar_hbm, acc_sh.at[pl.ds(s * RPW, RPW)])
    plsc.subcore_barrier()
    base = c * (EE // NC) + s * (EE // NC // NS)

    def body(i, carry):
        off = pl.multiple_of(base + i * K_NAR, 8)
        pltpu.sync_copy(fr_hbm.at[pl.ds(off, K_NAR)], fidx_v)
        pltpu.sync_copy(to_hbm.at[pl.ds(off, K_NAR)], tidx_v)
        pltpu.async_copy(dis_hbm.at[fidx_v], rows_v, sem).wait()
        pltpu.sync_copy(rows_v, acc_sh.at[tidx_v], add=True)
        return carry

    lax.fori_loop(0, NAR_CHUNKS, body, 0)
    plsc.subcore_barrier()
    pltpu.sync_copy(acc_sh.at[pl.ds(s * RPW, RPW)],
                    t_out.at[c, pl.ds(s * RPW, RPW)])


@functools.partial(
    pl.kernel,
    out_type=jax.ShapeDtypeStruct((NC, NPAD, HH), jnp.float32),
    mesh=_sc_mesh(),
    scratch_types=[
        pltpu.VMEM((K_ROW,), jnp.int32),
        pltpu.VMEM((K_ROW,), jnp.int32),
        pltpu.VMEM((K_ROW, HH), jnp.float32),
        pltpu.SemaphoreType.DMA,
        pltpu.VMEM_SHARED((NPAD, HH), jnp.float32),
    ],
)
def _row_pass(f2_hbm, to_hbm, ytab_hbm, zrow_hbm, agg_out,
              fidx_v, tidx_v, rows_v, sem, acc_sh):
    c = lax.axis_index("c")
    s = lax.axis_index("s")
    pltpu.sync_copy(zrow_hbm, acc_sh.at[pl.ds(s * RPW, RPW)])
    plsc.subcore_barrier()
    fbase = c * EE + s * (EE // NS)
    tbase = s * (EE // NS)

    def body(i, carry):
        foff = pl.multiple_of(fbase + i * K_ROW, 8)
        toff = pl.multiple_of(tbase + i * K_ROW, 8)
        pltpu.sync_copy(f2_hbm.at[pl.ds(foff, K_ROW)], fidx_v)
        pltpu.sync_copy(to_hbm.at[pl.ds(toff, K_ROW)], tidx_v)
        pltpu.async_copy(ytab_hbm.at[fidx_v], rows_v, sem).wait()
        pltpu.sync_copy(rows_v, acc_sh.at[tidx_v], add=True)
        return carry

    lax.fori_loop(0, ROW_CHUNKS, body, 0)
    plsc.subcore_barrier()
    pltpu.sync_copy(acc_sh.at[pl.ds(s * RPW, RPW)],
                    agg_out.at[c, pl.ds(s * RPW, RPW)])


def _prep_body(deg2_ref, emb_ref, dis_ref, y0_ref):
    deg = deg2_ref[0, :, 0:1] + deg2_ref[1, :, 0:1]
    dis = jnp.where(deg > 0.0, lax.rsqrt(deg), 0.0)
    dis_ref[...] = jnp.broadcast_to(dis, (BR, HH))
    y = dis * emb_ref[...]
    y0_ref[0] = y[:, :HH]
    y0_ref[1] = y[:, HH:]


def _prep_call(deg2, emb0):
    return pl.pallas_call(
        _prep_body,
        grid=(NN // BR,),
        in_specs=[
            pl.BlockSpec((NC, BR, HH), lambda i: (0, i, 0)),
            pl.BlockSpec((BR, DD), lambda i: (i, 0)),
        ],
        out_specs=[
            pl.BlockSpec((BR, HH), lambda i: (i, 0)),
            pl.BlockSpec((NC, BR, HH), lambda i: (0, i, 0)),
        ],
        out_shape=[
            jax.ShapeDtypeStruct((NN, HH), jnp.float32),
            jax.ShapeDtypeStruct((NC, NN, HH), jnp.float32),
        ],
    )(deg2, emb0)


def _layer_body(x_ref, agg_ref, dis_ref, t2_ref, w1_ref, w2_ref,
                b1_ref, b2_ref, xo_ref, y_ref):
    dis = dis_ref[:, 0:1]
    t = t2_ref[0, :, 0:1] + t2_ref[1, :, 0:1]
    sv = dis * t
    agg_raw = jnp.concatenate([agg_ref[0], agg_ref[1]], axis=-1)
    x = x_ref[...]
    agg1 = dis * agg_raw
    h1 = agg1 + x
    h2 = x * agg1
    out = (jnp.dot(h1, w1_ref[...], preferred_element_type=jnp.float32)
           + jnp.dot(h2, w2_ref[...], preferred_element_type=jnp.float32)
           + sv * (b1_ref[...] + b2_ref[...]) + b1_ref[...])
    xo = jnp.where(out >= 0.0, out, 0.01 * out)
    xo_ref[...] = xo
    y = dis * xo
    y_ref[0] = y[:, :HH]
    y_ref[1] = y[:, HH:]


def _layer_call(x, agg, dis_b, t2, w1t, w2t, b1l, b2l):
    return pl.pallas_call(
        _layer_body,
        grid=(NN // BR,),
        in_specs=[
            pl.BlockSpec((BR, DD), lambda i: (i, 0)),
            pl.BlockSpec((NC, BR, HH), lambda i: (0, i, 0)),
            pl.BlockSpec((BR, HH), lambda i: (i, 0)),
            pl.BlockSpec((NC, BR, HH), lambda i: (0, i, 0)),
            pl.BlockSpec((DD, DD), lambda i: (0, 0)),
            pl.BlockSpec((DD, DD), lambda i: (0, 0)),
            pl.BlockSpec((1, DD), lambda i: (0, 0)),
            pl.BlockSpec((1, DD), lambda i: (0, 0)),
        ],
        out_specs=[
            pl.BlockSpec((BR, DD), lambda i: (i, 0)),
            pl.BlockSpec((NC, BR, HH), lambda i: (0, i, 0)),
        ],
        out_shape=[
            jax.ShapeDtypeStruct((NN, DD), jnp.float32),
            jax.ShapeDtypeStruct((NC, NN, HH), jnp.float32),
        ],
    )(x, agg, dis_b, t2, w1t, w2t, b1l, b2l)


def kernel(edge_index, edge_attrs, emb_weight, W1, b1, W2, b2):
    fr = edge_index[0]
    to = edge_index[1]
    f2 = jnp.concatenate([fr, fr + NN])
    ones_nar = jnp.ones((K_NAR, HH), jnp.float32)
    zrow = jnp.zeros((RPW, HH), jnp.float32)

    deg2 = _deg_pass(to, ones_nar, zrow)
    dis_b, y0 = _prep_call(deg2, emb_weight)
    t2 = _t_pass(fr, to, dis_b, zrow)

    x = emb_weight
    embs = [emb_weight]
    y = y0
    for l in range(LL):
        agg = _row_pass(f2, to, y.reshape(NC * NN, HH), zrow)
        x, y = _layer_call(x, agg, dis_b, t2, W1[l].T, W2[l].T,
                           b1[l][None, :], b2[l][None, :])
        embs.append(x)
    out = jnp.concatenate(embs, axis=-1)
    return emb_weight, out
